# baseline (speedup 1.0000x reference)
import functools
import math

import jax
import jax.numpy as jnp
from jax import lax
from jax.experimental import pallas as pl
from jax.experimental.pallas import tpu as pltpu
from jax.experimental.pallas import tpu_sc as plsc

DIM = 512
PIX = DIM * DIM
NC, NS, L = 2, 16, 16
NW = NC * NS
B_PER_W = PIX // NW
NBLK = B_PER_W // L


def _affine_matrix():
    key = jax.random.key(42)
    k = jax.random.split(key, 6)
    rot = 15.0 * jax.random.normal(k[0], [1], dtype=jnp.float32)
    shr = 5.0 * jax.random.normal(k[1], [1], dtype=jnp.float32)
    h_zoom = 1.0 + jax.random.normal(k[2], [1], dtype=jnp.float32) / 10.0
    w_zoom = 1.0 + jax.random.normal(k[3], [1], dtype=jnp.float32) / 10.0
    h_shift = 16.0 * jax.random.normal(k[4], [1], dtype=jnp.float32)
    w_shift = 16.0 * jax.random.normal(k[5], [1], dtype=jnp.float32)

    rotation = math.pi * rot / 180.0
    shear = math.pi * shr / 180.0
    c1 = jnp.cos(rotation)
    s1 = jnp.sin(rotation)
    one = jnp.ones([1], dtype=jnp.float32)
    zero = jnp.zeros([1], dtype=jnp.float32)
    rot_m = jnp.reshape(
        jnp.concatenate([c1, s1, zero, -s1, c1, zero, zero, zero, one], 0), [3, 3]
    )
    c2 = jnp.cos(shear)
    s2 = jnp.sin(shear)
    shear_m = jnp.reshape(
        jnp.concatenate([one, s2, zero, zero, c2, zero, zero, zero, one], 0), [3, 3]
    )
    zoom_m = jnp.reshape(
        jnp.concatenate(
            [one / h_zoom, zero, zero, zero, one / w_zoom, zero, zero, zero, one], 0
        ),
        [3, 3],
    )
    shift_m = jnp.reshape(
        jnp.concatenate([one, zero, h_shift, zero, one, w_shift, zero, zero, one], 0),
        [3, 3],
    )
    return jnp.dot(jnp.dot(rot_m, shear_m), jnp.dot(zoom_m, shift_m))


_MESH = plsc.VectorSubcoreMesh(
    core_axis_name="c", subcore_axis_name="s", num_cores=NC, num_subcores=NS
)


@functools.partial(
    pl.kernel,
    out_type=jax.ShapeDtypeStruct((PIX, 3), jnp.float32),
    mesh=_MESH,
    scratch_types=[
        pltpu.VMEM((B_PER_W,), jnp.int32),
        pltpu.VMEM((B_PER_W, 3), jnp.float32),
        pltpu.VMEM((6, L), jnp.float32),
        pltpu.SemaphoreType.DMA,
    ],
    compiler_params=pltpu.CompilerParams(use_tc_tiling_on_sc=False),
)
def _warp(table_hbm, coef_hbm, out_hbm, idx_v, rows_v, coef_v, sem):
    wid = lax.axis_index("s") * NC + lax.axis_index("c")
    base = wid * B_PER_W

    pltpu.sync_copy(coef_hbm, coef_v)
    a0 = coef_v[0]
    a1 = coef_v[1]
    a2 = coef_v[2]
    b0 = coef_v[3]
    b1 = coef_v[4]
    b2 = coef_v[5]
    lane = lax.iota(jnp.int32, L)

    def body(t, carry):
        p = base + t * L + lane
        i = lax.shift_right_logical(p, 9)
        j = lax.bitwise_and(p, DIM - 1)
        xs = (DIM // 2 - i).astype(jnp.float32)
        ys = (j - DIM // 2).astype(jnp.float32)
        t0 = a0 * xs + a1 * ys + a2
        t1 = b0 * xs + b1 * ys + b2
        r = DIM // 2 - jnp.clip(t0.astype(jnp.int32), -(DIM // 2) + 1, DIM // 2)
        q = DIM // 2 - 1 + jnp.clip(t1.astype(jnp.int32), -(DIM // 2) + 1, DIM // 2)
        idx_v[pl.ds(t * L, L)] = r * DIM + q
        return carry

    lax.fori_loop(0, NBLK, body, 0)

    pltpu.async_copy(table_hbm.at[idx_v], rows_v, sem).wait()
    pltpu.sync_copy(rows_v, out_hbm.at[pl.ds(base, B_PER_W)])


def kernel(x):
    table = x.reshape(PIX, 3)
    m = _affine_matrix()
    coefs = jnp.broadcast_to(jnp.reshape(m[:2].reshape(6), (6, 1)), (6, L))
    out = _warp(table, coefs)
    return out.reshape(DIM, DIM, 3)
